# Initial kernel scaffold; baseline (speedup 1.0000x reference)
import jax
import jax.numpy as jnp
from jax.experimental import pallas as pl


def kernel(inputs, token_table, pos_table):
    raise NotImplementedError("write your pallas kernel here")


# speedup vs baseline: 3.3475x; 3.3475x over previous
import functools

import jax
import jax.numpy as jnp
from jax import lax
from jax.experimental import pallas as pl
from jax.experimental.pallas import tpu as pltpu
from jax.experimental.pallas import tpu_sc as plsc

VOCAB = 100000
MAX_LEN = 200
EMBED = 64
BATCH = 4096

NC = 2
NS = 16
NW = NC * NS
LANES = 16

K_ROWS = 2
CHUNK = K_ROWS * MAX_LEN
G = 4
GI = CHUNK // G
ROWS_PER_W = BATCH // NW
NCHUNK = ROWS_PER_W // K_ROWS
IDX_ROWS_PER_CHUNK = CHUNK // GI


def _body(idx_hbm, tok_hbm, pos_hbm, out_hbm, idx_v, rows_v, pos_v, gsem, osem):
    wid = lax.axis_index("s") * NC + lax.axis_index("c")

    pltpu.sync_copy(pos_hbm, pos_v.at[pl.ds(0, MAX_LEN)])
    pltpu.sync_copy(pos_hbm, pos_v.at[pl.ds(MAX_LEN, MAX_LEN)])

    idx_row0 = wid * (ROWS_PER_W * MAX_LEN // GI)
    out_row0 = wid * (ROWS_PER_W * MAX_LEN)

    def chunk_body(c, carry):
        pltpu.sync_copy(idx_hbm.at[pl.ds(idx_row0 + c * IDX_ROWS_PER_CHUNK,
                                         IDX_ROWS_PER_CHUNK)], idx_v)
        for j in range(G):
            pltpu.async_copy(tok_hbm.at[idx_v.at[j]],
                             rows_v.at[pl.ds(j * GI, GI)], gsem)
        for j in range(G):
            pltpu.make_async_copy(tok_hbm.at[idx_v.at[j]],
                                  rows_v.at[pl.ds(j * GI, GI)], gsem).wait()

        def add_body(i, _):
            for j in range(EMBED // LANES):
                s = pl.ds(j * LANES, LANES)
                rows_v[i, s] = rows_v[i, s] + pos_v[i, s]
            return _
        lax.fori_loop(0, CHUNK, add_body, 0)

        pltpu.async_copy(rows_v, out_hbm.at[pl.ds(out_row0 + c * CHUNK, CHUNK)],
                         osem)
        pltpu.make_async_copy(rows_v, out_hbm.at[pl.ds(out_row0 + c * CHUNK, CHUNK)],
                              osem).wait()
        return carry

    lax.fori_loop(0, NCHUNK, chunk_body, 0)


def kernel(inputs, token_table, pos_table):
    idx2d = inputs.astype(jnp.int32).reshape(BATCH * MAX_LEN // GI, GI)
    mesh = plsc.VectorSubcoreMesh(core_axis_name="c", subcore_axis_name="s")
    run = functools.partial(
        pl.kernel,
        mesh=mesh,
        compiler_params=pltpu.CompilerParams(use_tc_tiling_on_sc=False),
        out_type=jax.ShapeDtypeStruct((BATCH * MAX_LEN, EMBED), jnp.float32),
        scratch_types=[
            pltpu.VMEM((IDX_ROWS_PER_CHUNK, GI), jnp.int32),
            pltpu.VMEM((CHUNK, EMBED), jnp.float32),
            pltpu.VMEM((CHUNK, EMBED), jnp.float32),
            pltpu.SemaphoreType.DMA,
            pltpu.SemaphoreType.DMA,
        ],
    )(_body)
    out = run(idx2d, token_table, pos_table)
    return out.reshape(BATCH, MAX_LEN, EMBED)


# speedup vs baseline: 4.2122x; 1.2583x over previous
import functools

import jax
import jax.numpy as jnp
from jax import lax
from jax.experimental import pallas as pl
from jax.experimental.pallas import tpu as pltpu
from jax.experimental.pallas import tpu_sc as plsc

VOCAB = 100000
MAX_LEN = 200
EMBED = 64
BATCH = 4096

NC = 2
NS = 16
NW = NC * NS
LANES = 16

K_ROWS = 2
CHUNK = K_ROWS * MAX_LEN
G = 4
GI = CHUNK // G
ROWS_PER_W = BATCH // NW
NCHUNK = ROWS_PER_W // K_ROWS
IDX_ROWS = CHUNK // GI
NBUF = 4


def _body(idx_hbm, tok_hbm, pos_hbm, out_hbm, idx_v, rows_v, pos_v, *sems):
    isem = sems[0:NBUF]
    gsem = sems[NBUF:2 * NBUF]
    osem = sems[2 * NBUF:3 * NBUF]

    wid = lax.axis_index("s") * NC + lax.axis_index("c")
    idx_row0 = wid * (ROWS_PER_W * MAX_LEN // GI)
    brow0 = wid * ROWS_PER_W

    pltpu.sync_copy(pos_hbm, pos_v)

    def fetch_idx(c, b):
        pltpu.async_copy(
            idx_hbm.at[pl.ds(idx_row0 + c * IDX_ROWS, IDX_ROWS)],
            idx_v.at[b], isem[b])

    def start_gather(c, b):
        pltpu.make_async_copy(
            idx_hbm.at[pl.ds(idx_row0 + c * IDX_ROWS, IDX_ROWS)],
            idx_v.at[b], isem[b]).wait()
        for j in range(G):
            pltpu.async_copy(
                tok_hbm.at[idx_v.at[b, j]],
                rows_v.at[b, j // 2, pl.ds((j % 2) * GI, GI)], gsem[b])

    def wait_out(c, b):
        pltpu.make_async_copy(
            rows_v.at[b],
            out_hbm.at[pl.ds(brow0 + K_ROWS * c, K_ROWS)], osem[b]).wait()

    def finish(c, b):
        for j in range(G):
            pltpu.make_async_copy(
                tok_hbm.at[idx_v.at[b, j]],
                rows_v.at[b, j // 2, pl.ds((j % 2) * GI, GI)], gsem[b]).wait()

        def add_body(l, carry):
            for r in range(K_ROWS):
                for jj in range(EMBED // LANES):
                    s = pl.ds(jj * LANES, LANES)
                    rows_v[b, r, l, s] = rows_v[b, r, l, s] + pos_v[l, s]
            return carry
        lax.fori_loop(0, MAX_LEN, add_body, 0)

        pltpu.async_copy(
            rows_v.at[b],
            out_hbm.at[pl.ds(brow0 + K_ROWS * c, K_ROWS)], osem[b])

    for c in range(NBUF - 1):
        fetch_idx(c, c)
    start_gather(0, 0)
    start_gather(1, 1)

    def step(s, carry):
        for b in range(NBUF):
            c = NBUF * s + b
            cf = c + NBUF - 1
            cg = c + NBUF - 2
            bf = (b + NBUF - 1) % NBUF
            bg = (b + NBUF - 2) % NBUF

            @pl.when(cf < NCHUNK)
            def _():
                fetch_idx(cf, bf)

            @pl.when(cg < NCHUNK)
            def _():
                @pl.when(cg >= NBUF)
                def _():
                    wait_out(cg - NBUF, bg)
                start_gather(cg, bg)

            finish(c, b)
        return carry

    lax.fori_loop(0, NCHUNK // NBUF, step, 0)

    for b in range(NBUF):
        wait_out(NCHUNK - NBUF + b, b)


def kernel(inputs, token_table, pos_table):
    idx2d = inputs.astype(jnp.int32).reshape(BATCH * MAX_LEN // GI, GI)
    mesh = plsc.VectorSubcoreMesh(core_axis_name="c", subcore_axis_name="s")
    run = functools.partial(
        pl.kernel,
        mesh=mesh,
        compiler_params=pltpu.CompilerParams(use_tc_tiling_on_sc=False),
        out_type=jax.ShapeDtypeStruct((BATCH, MAX_LEN, EMBED), jnp.float32),
        scratch_types=[
            pltpu.VMEM((NBUF, IDX_ROWS, GI), jnp.int32),
            pltpu.VMEM((NBUF, K_ROWS, MAX_LEN, EMBED), jnp.float32),
            pltpu.VMEM((MAX_LEN, EMBED), jnp.float32),
        ] + [pltpu.SemaphoreType.DMA] * (3 * NBUF),
    )(_body)
    return run(idx2d, token_table, pos_table)
